# baseline (speedup 1.0000x reference)
import functools

import jax
import jax.numpy as jnp
from jax.experimental import pallas as pl
from jax.experimental.pallas import tpu as pltpu
from jax.experimental.pallas import tpu_sc as plsc

NPOINT = 512
RADIUS = 0.4
K = 64
SUB = 8


def _fps_kernel(xyzr_ref, xs_ref, ys_ref, zs_ref, idx_ref, new_ref, *dists_refs):
    nb = xyzr_ref.shape[0]
    ncol = dists_refs[0].shape[1]
    n = SUB * ncol
    for b in range(nb):
        dists_refs[b][...] = jnp.full((SUB, ncol), 1e10, dtype=jnp.float32)
    flat_iota = (
        jax.lax.broadcasted_iota(jnp.int32, (SUB, ncol), 0) * ncol
        + jax.lax.broadcasted_iota(jnp.int32, (SUB, ncol), 1)
    )

    def body(i, fars):
        dnews = []
        crows = []
        for b in range(nb):
            crow = xyzr_ref[b, fars[b], :]
            crows.append(crow)
            cx = jnp.broadcast_to(crow[0:1].reshape(1, 1), (SUB, ncol))
            cy = jnp.broadcast_to(crow[1:2].reshape(1, 1), (SUB, ncol))
            cz = jnp.broadcast_to(crow[2:3].reshape(1, 1), (SUB, ncol))
            dx = xs_ref[b] - cx
            dy = ys_ref[b] - cy
            dz = zs_ref[b] - cz
            d = dx * dx + dy * dy + dz * dz
            dnews.append(jnp.minimum(dists_refs[b][...], d))
        ms = [jnp.max(dnews[b]) for b in range(nb)]
        for b in range(nb):
            dists_refs[b][...] = dnews[b]
            idx_ref[b, 0, i] = fars[b]
            new_ref[b, i, :] = crows[b]
        cands = [jnp.where(dnews[b] == ms[b], flat_iota, n) for b in range(nb)]
        return tuple(jnp.min(cands[b]).astype(jnp.int32) for b in range(nb))

    jax.lax.fori_loop(0, NPOINT, body, (jnp.int32(0),) * nb)


def _fps_call(points_xyz):
    b, n, _ = points_xyz.shape
    ngrp = 2 if b % 2 == 0 else 1
    nb = b // ngrp
    xs = points_xyz[..., 0].reshape(b, SUB, n // SUB)
    ys = points_xyz[..., 1].reshape(b, SUB, n // SUB)
    zs = points_xyz[..., 2].reshape(b, SUB, n // SUB)
    idx, new_xyz = pl.pallas_call(
        _fps_kernel,
        grid=(ngrp,),
        in_specs=[
            pl.BlockSpec((nb, n, 3), lambda i: (i, 0, 0)),
            pl.BlockSpec((nb, SUB, n // SUB), lambda i: (i, 0, 0)),
            pl.BlockSpec((nb, SUB, n // SUB), lambda i: (i, 0, 0)),
            pl.BlockSpec((nb, SUB, n // SUB), lambda i: (i, 0, 0)),
        ],
        out_specs=[
            pl.BlockSpec((nb, 1, NPOINT), lambda i: (i, 0, 0), memory_space=pltpu.SMEM),
            pl.BlockSpec((nb, NPOINT, 3), lambda i: (i, 0, 0)),
        ],
        out_shape=[
            jax.ShapeDtypeStruct((b, 1, NPOINT), jnp.int32),
            jax.ShapeDtypeStruct((b, NPOINT, 3), jnp.float32),
        ],
        scratch_shapes=[pltpu.VMEM((SUB, n // SUB), jnp.float32) for _ in range(nb)],
        compiler_params=pltpu.CompilerParams(dimension_semantics=("parallel",)),
    )(points_xyz, xs, ys, zs)
    return idx.reshape(b, NPOINT), new_xyz


_NC = 512


def _ballquery_kernel(newxyz_ref, xr_ref, yr_ref, zr_ref, gidx_ref):
    nchunks = xr_ref.shape[1]
    m = newxyz_ref.shape[1]
    r2 = RADIUS * RADIUS
    cx = newxyz_ref[0, :, 0:1]
    cy = newxyz_ref[0, :, 1:2]
    cz = newxyz_ref[0, :, 2:3]
    tri = jnp.where(
        jax.lax.broadcasted_iota(jnp.int32, (_NC, _NC), 0)
        <= jax.lax.broadcasted_iota(jnp.int32, (_NC, _NC), 1),
        1.0,
        0.0,
    ).astype(jnp.bfloat16)
    ones_nc = jnp.ones((_NC, 1), dtype=jnp.bfloat16)

    def cond(carry):
        t, _, base = carry
        return jnp.logical_and(t < nchunks, jnp.min(base) < jnp.float32(K))

    def chunk_body(carry):
        t, acc, base = carry
        px = xr_ref[0, pl.ds(t, 1), :]
        py = yr_ref[0, pl.ds(t, 1), :]
        pz = zr_ref[0, pl.ds(t, 1), :]
        dx = cx - px
        dy = cy - py
        dz = cz - pz
        d2 = dx * dx + dy * dy + dz * dz
        valid = jnp.where(d2 < r2, 1.0, 0.0).astype(jnp.bfloat16)
        s = (
            jax.lax.dot_general(
                valid, tri, (((1,), (0,)), ((), ())),
                preferred_element_type=jnp.float32,
            )
            + base
        )
        sbf = s.astype(jnp.bfloat16)
        cols = []
        for j in range(K):
            mask = jnp.where(
                sbf <= jnp.bfloat16(j), jnp.bfloat16(1), jnp.bfloat16(0)
            )
            cols.append(
                jax.lax.dot_general(
                    mask, ones_nc, (((1,), (0,)), ((), ())),
                    preferred_element_type=jnp.float32,
                )
            )
        return t + 1, acc + jnp.concatenate(cols, axis=1), s[:, _NC - 1 : _NC]

    _, acc, total = jax.lax.while_loop(
        cond,
        chunk_body,
        (
            jnp.int32(0),
            jnp.zeros((m, K), dtype=jnp.float32),
            jnp.zeros((m, 1), dtype=jnp.float32),
        ),
    )
    raw = acc
    first = raw[:, 0:1]
    first = jnp.where(total >= 1.0, first, 0.0)
    jcol = jax.lax.broadcasted_iota(jnp.int32, (m, K), 1).astype(jnp.float32)
    gidx = jnp.where(jcol < total, raw, first)
    gidx_ref[0] = gidx.astype(jnp.int32)


def _ballquery_call(points_xyz, new_xyz):
    b, n, _ = points_xyz.shape
    nchunks = n // _NC
    xr = points_xyz[..., 0].reshape(b, nchunks, _NC)
    yr = points_xyz[..., 1].reshape(b, nchunks, _NC)
    zr = points_xyz[..., 2].reshape(b, nchunks, _NC)
    gidx = pl.pallas_call(
        _ballquery_kernel,
        grid=(b,),
        in_specs=[
            pl.BlockSpec((1, NPOINT, 3), lambda i: (i, 0, 0)),
            pl.BlockSpec((1, nchunks, _NC), lambda i: (i, 0, 0)),
            pl.BlockSpec((1, nchunks, _NC), lambda i: (i, 0, 0)),
            pl.BlockSpec((1, nchunks, _NC), lambda i: (i, 0, 0)),
        ],
        out_specs=pl.BlockSpec((1, NPOINT, K), lambda i: (i, 0, 0)),
        out_shape=jax.ShapeDtypeStruct((b, NPOINT, K), jnp.int32),
        compiler_params=pltpu.CompilerParams(dimension_semantics=("parallel",)),
    )(new_xyz, xr, yr, zr)
    return gidx


_ZT = 2048


def _buildz_kernel(f_ref, xyz_ref, w0f_ref, w0x_ref, b0_ref, z_ref):
    fpart = jax.lax.dot_general(
        f_ref[0], w0f_ref[...], (((0,), (1,)), ((), ())),
        preferred_element_type=jnp.float32,
    )
    xpart = jax.lax.dot_general(
        xyz_ref[0], w0x_ref[...], (((1,), (1,)), ((), ())),
        preferred_element_type=jnp.float32,
    )
    z_ref[0] = fpart + xpart + b0_ref[...]


def _buildz_call(features, points_xyz, W0, b0):
    b, c, n = features.shape
    o = W0.shape[0]
    w0x = W0[:, :3]
    w0f = W0[:, 3:]
    z = pl.pallas_call(
        _buildz_kernel,
        grid=(b, n // _ZT),
        in_specs=[
            pl.BlockSpec((1, c, _ZT), lambda i, t: (i, 0, t)),
            pl.BlockSpec((1, _ZT, 3), lambda i, t: (i, t, 0)),
            pl.BlockSpec((o, c), lambda i, t: (0, 0)),
            pl.BlockSpec((o, 3), lambda i, t: (0, 0)),
            pl.BlockSpec((1, o), lambda i, t: (0, 0)),
        ],
        out_specs=pl.BlockSpec((1, _ZT, o), lambda i, t: (i, t, 0)),
        out_shape=jax.ShapeDtypeStruct((b, n, o), jnp.float32),
        compiler_params=pltpu.CompilerParams(
            dimension_semantics=("parallel", "parallel")
        ),
    )(features, points_xyz, w0f, w0x, b0.reshape(1, o))
    return z.reshape(b * n, o)


_GW = 128


def _sc_gather(table, flat_idx):
    num = flat_idx.shape[0]
    o = table.shape[1]
    idx2 = flat_idx.reshape(1, num)
    mesh = plsc.VectorSubcoreMesh(core_axis_name="c", subcore_axis_name="s")

    @functools.partial(
        pl.kernel,
        out_type=jax.ShapeDtypeStruct((num, o), table.dtype),
        mesh=mesh,
    )
    def gather_kernel(x_hbm, i_hbm, o_hbm):
        def body(i_vmem, o_vmem):
            pltpu.sync_copy(x_hbm.at[i_vmem.at[0]], o_vmem)

        pltpu.emit_pipeline(
            body,
            grid=(num // _GW,),
            in_specs=[pl.BlockSpec((1, _GW), index_map=lambda i: (0, i))],
            out_specs=[pl.BlockSpec((_GW, o), index_map=lambda i: (i, 0))],
            core_axis_name=("c", "s"),
            dimension_semantics=(pltpu.PARALLEL,),
        )(i_hbm, o_hbm)

    return gather_kernel(table, idx2)


_MT = 64


def _mlp_kernel(g_ref, nxyz_ref, w0x_ref, w1t_ref, b1_ref, w2t_ref, b2_ref, out_ref):
    o0 = g_ref.shape[1]
    o2 = w2t_ref.shape[1]
    cterm = jax.lax.dot_general(
        nxyz_ref[0], w0x_ref[...], (((1,), (1,)), ((), ())),
        preferred_element_type=jnp.float32,
    )
    g3 = g_ref[...].reshape(_MT, K, o0)
    h0 = jnp.maximum(
        g3 - jnp.broadcast_to(cterm.reshape(_MT, 1, o0), (_MT, K, o0)), 0.0
    )
    h0f = h0.reshape(_MT * K, o0).astype(jnp.bfloat16)
    h1 = jnp.maximum(
        jax.lax.dot_general(
            h0f, w1t_ref[...], (((1,), (0,)), ((), ())),
            preferred_element_type=jnp.float32,
        )
        + b1_ref[...],
        0.0,
    ).astype(jnp.bfloat16)
    h2 = jnp.maximum(
        jax.lax.dot_general(
            h1, w2t_ref[...], (((1,), (0,)), ((), ())),
            preferred_element_type=jnp.float32,
        )
        + b2_ref[...],
        0.0,
    )
    pooled = jnp.max(h2.reshape(_MT, K, o2), axis=1)
    out_ref[0] = pooled


def _mlp_call(gathered, new_xyz, W0, W1, b1, W2, b2):
    b = new_xyz.shape[0]
    o0 = W0.shape[0]
    o1 = W1.shape[0]
    o2 = W2.shape[0]
    w0x = W0[:, :3]
    w1t = W1.T.astype(jnp.bfloat16)
    w2t = W2.T.astype(jnp.bfloat16)
    tiles = NPOINT // _MT
    pooled = pl.pallas_call(
        _mlp_kernel,
        grid=(b, tiles),
        in_specs=[
            pl.BlockSpec((_MT * K, o0), lambda i, t: (i * tiles + t, 0)),
            pl.BlockSpec((1, _MT, 3), lambda i, t: (i, t, 0)),
            pl.BlockSpec((o0, 3), lambda i, t: (0, 0)),
            pl.BlockSpec((o0, o1), lambda i, t: (0, 0)),
            pl.BlockSpec((1, o1), lambda i, t: (0, 0)),
            pl.BlockSpec((o1, o2), lambda i, t: (0, 0)),
            pl.BlockSpec((1, o2), lambda i, t: (0, 0)),
        ],
        out_specs=pl.BlockSpec((1, _MT, o2), lambda i, t: (i, t, 0)),
        out_shape=jax.ShapeDtypeStruct((b, NPOINT, o2), jnp.float32),
        compiler_params=pltpu.CompilerParams(
            dimension_semantics=("parallel", "parallel")
        ),
    )(
        gathered,
        new_xyz,
        w0x,
        w1t,
        b1.reshape(1, o1),
        w2t,
        b2.reshape(1, o2),
    )
    return pooled


def kernel(points_xyz, features, W0, b0, W1, b1, W2, b2):
    b, n, _ = points_xyz.shape
    idx, new_xyz = _fps_call(points_xyz)
    gidx = _ballquery_call(points_xyz, new_xyz)
    z = _buildz_call(features, points_xyz, W0, b0)
    offs = (jnp.arange(b, dtype=jnp.int32) * n).reshape(b, 1, 1)
    flat_idx = (gidx + offs).reshape(b * NPOINT * K)
    gathered = _sc_gather(z, flat_idx)
    pooled = _mlp_call(gathered, new_xyz, W0, W1, b1, W2, b2)
    new_features = jnp.transpose(pooled, (0, 2, 1))
    return (new_xyz, new_features, idx)


# speedup vs baseline: 1.4071x; 1.4071x over previous
import functools

import jax
import jax.numpy as jnp
from jax.experimental import pallas as pl
from jax.experimental.pallas import tpu as pltpu
from jax.experimental.pallas import tpu_sc as plsc

NPOINT = 512
RADIUS = 0.4
K = 64
SUB = 8


def _fps_kernel(xyzh_ref, xs_ref, ys_ref, zs_ref, idx_ref, new_ref, *scratch):
    xyzs_ref = scratch[0]
    dma_sem = scratch[1]
    dists_refs = scratch[2:]
    pltpu.make_async_copy(xyzh_ref, xyzs_ref, dma_sem).start()
    pltpu.make_async_copy(xyzh_ref, xyzs_ref, dma_sem).wait()
    _fps_body(xyzs_ref, xs_ref, ys_ref, zs_ref, idx_ref, new_ref, dists_refs)


def _fps_body(xyzs_ref, xs_ref, ys_ref, zs_ref, idx_ref, new_ref, dists_refs):
    nb = xs_ref.shape[0]
    ncol = dists_refs[0].shape[1]
    xs_ref, ys_ref, zs_ref = xs_ref.at[:, 0], ys_ref.at[:, 0], zs_ref.at[:, 0]
    n = SUB * ncol
    for b in range(nb):
        dists_refs[b][...] = jnp.full((SUB, ncol), 1e10, dtype=jnp.float32)
    flat_iota = (
        jax.lax.broadcasted_iota(jnp.int32, (SUB, ncol), 0) * ncol
        + jax.lax.broadcasted_iota(jnp.int32, (SUB, ncol), 1)
    )

    def body(i, fars):
        dnews = []
        for b in range(nb):
            far = fars[b]
            cx = xyzs_ref[b, 3 * far]
            cy = xyzs_ref[b, 3 * far + 1]
            cz = xyzs_ref[b, 3 * far + 2]
            idx_ref[b, 0, i] = far
            new_ref[b, 0, 3 * i] = cx
            new_ref[b, 0, 3 * i + 1] = cy
            new_ref[b, 0, 3 * i + 2] = cz
            dx = xs_ref[b] - cx
            dy = ys_ref[b] - cy
            dz = zs_ref[b] - cz
            d = dx * dx + dy * dy + dz * dz
            dnews.append(jnp.minimum(dists_refs[b][...], d))
        ms = [jnp.max(dnews[b]) for b in range(nb)]
        for b in range(nb):
            dists_refs[b][...] = dnews[b]
        cands = [jnp.where(dnews[b] == ms[b], flat_iota, n) for b in range(nb)]
        return tuple(jnp.min(cands[b]).astype(jnp.int32) for b in range(nb))

    jax.lax.fori_loop(0, NPOINT, body, (jnp.int32(0),) * nb)


def _fps_call(points_xyz, xyz_t):
    b, n, _ = points_xyz.shape
    xt4 = xyz_t.reshape(b, 3, SUB, n // SUB)
    idx, new_xyz = pl.pallas_call(
        _fps_kernel,
        grid=(1,),
        in_specs=[
            pl.BlockSpec(memory_space=pl.ANY),
            pl.BlockSpec((b, 1, SUB, n // SUB), lambda g: (0, 0, 0, 0)),
            pl.BlockSpec((b, 1, SUB, n // SUB), lambda g: (0, 1, 0, 0)),
            pl.BlockSpec((b, 1, SUB, n // SUB), lambda g: (0, 2, 0, 0)),
        ],
        out_specs=[
            pl.BlockSpec((b, 1, NPOINT), lambda g: (0, 0, 0), memory_space=pltpu.SMEM),
            pl.BlockSpec(
                (b, 1, 3 * NPOINT), lambda g: (0, 0, 0), memory_space=pltpu.SMEM
            ),
        ],
        out_shape=[
            jax.ShapeDtypeStruct((b, 1, NPOINT), jnp.int32),
            jax.ShapeDtypeStruct((b, 1, 3 * NPOINT), jnp.float32),
        ],
        scratch_shapes=[pltpu.SMEM((b, 3 * n), jnp.float32), pltpu.SemaphoreType.DMA]
        + [pltpu.VMEM((SUB, n // SUB), jnp.float32) for _ in range(b)],
    )(points_xyz.reshape(b, 3 * n), xt4, xt4, xt4)
    return idx.reshape(b, NPOINT), new_xyz.reshape(b, NPOINT, 3)


_NC = 512


def _ballquery_kernel(newxyz_ref, xr_ref, yr_ref, zr_ref, gidx_ref):
    xr_ref, yr_ref, zr_ref = xr_ref.at[:, 0], yr_ref.at[:, 0], zr_ref.at[:, 0]
    nchunks = xr_ref.shape[1]
    m = newxyz_ref.shape[1]
    r2 = RADIUS * RADIUS
    cx = newxyz_ref[0, :, 0:1]
    cy = newxyz_ref[0, :, 1:2]
    cz = newxyz_ref[0, :, 2:3]
    tri = jnp.where(
        jax.lax.broadcasted_iota(jnp.int32, (_NC, _NC), 0)
        <= jax.lax.broadcasted_iota(jnp.int32, (_NC, _NC), 1),
        1.0,
        0.0,
    ).astype(jnp.bfloat16)
    ones_nc = jnp.ones((_NC, 1), dtype=jnp.bfloat16)

    def cond(carry):
        t, _, base = carry
        return jnp.logical_and(t < nchunks, jnp.min(base) < jnp.float32(K))

    def chunk_body(carry):
        t, acc, base = carry
        px = xr_ref[0, pl.ds(t, 1), :]
        py = yr_ref[0, pl.ds(t, 1), :]
        pz = zr_ref[0, pl.ds(t, 1), :]
        dx = cx - px
        dy = cy - py
        dz = cz - pz
        d2 = dx * dx + dy * dy + dz * dz
        valid = jnp.where(d2 < r2, 1.0, 0.0).astype(jnp.bfloat16)
        s = (
            jax.lax.dot_general(
                valid, tri, (((1,), (0,)), ((), ())),
                preferred_element_type=jnp.float32,
            )
            + base
        )
        sbf = s.astype(jnp.bfloat16)
        cols = []
        for j in range(K):
            mask = jnp.where(
                sbf <= jnp.bfloat16(j), jnp.bfloat16(1), jnp.bfloat16(0)
            )
            cols.append(
                jax.lax.dot_general(
                    mask, ones_nc, (((1,), (0,)), ((), ())),
                    preferred_element_type=jnp.float32,
                )
            )
        return t + 1, acc + jnp.concatenate(cols, axis=1), s[:, _NC - 1 : _NC]

    _, acc, total = jax.lax.while_loop(
        cond,
        chunk_body,
        (
            jnp.int32(0),
            jnp.zeros((m, K), dtype=jnp.float32),
            jnp.zeros((m, 1), dtype=jnp.float32),
        ),
    )
    raw = acc
    first = raw[:, 0:1]
    first = jnp.where(total >= 1.0, first, 0.0)
    jcol = jax.lax.broadcasted_iota(jnp.int32, (m, K), 1).astype(jnp.float32)
    gidx = jnp.where(jcol < total, raw, first)
    gidx_ref[0] = gidx.astype(jnp.int32)


def _ballquery_call(points_xyz, new_xyz, xyz_t):
    b, n, _ = points_xyz.shape
    nchunks = n // _NC
    xt4 = xyz_t.reshape(b, 3, nchunks, _NC)
    gidx = pl.pallas_call(
        _ballquery_kernel,
        grid=(b,),
        in_specs=[
            pl.BlockSpec((1, NPOINT, 3), lambda i: (i, 0, 0)),
            pl.BlockSpec((1, 1, nchunks, _NC), lambda i: (i, 0, 0, 0)),
            pl.BlockSpec((1, 1, nchunks, _NC), lambda i: (i, 1, 0, 0)),
            pl.BlockSpec((1, 1, nchunks, _NC), lambda i: (i, 2, 0, 0)),
        ],
        out_specs=pl.BlockSpec((1, NPOINT, K), lambda i: (i, 0, 0)),
        out_shape=jax.ShapeDtypeStruct((b, NPOINT, K), jnp.int32),
    )(new_xyz, xt4, xt4, xt4)
    return gidx


_ZT = 2048


def _buildz_kernel(f_ref, xyz_ref, w0f_ref, w0x_ref, b0_ref, z_ref):
    fpart = jax.lax.dot_general(
        f_ref[0], w0f_ref[...], (((0,), (1,)), ((), ())),
        preferred_element_type=jnp.float32,
    )
    xpart = jax.lax.dot_general(
        xyz_ref[0], w0x_ref[...], (((1,), (1,)), ((), ())),
        preferred_element_type=jnp.float32,
    )
    z_ref[0] = fpart + xpart + b0_ref[...]


def _buildz_call(features, points_xyz, W0, b0):
    b, c, n = features.shape
    o = W0.shape[0]
    w0x = W0[:, :3]
    w0f = W0[:, 3:]
    z = pl.pallas_call(
        _buildz_kernel,
        grid=(b, n // _ZT),
        in_specs=[
            pl.BlockSpec((1, c, _ZT), lambda i, t: (i, 0, t)),
            pl.BlockSpec((1, _ZT, 3), lambda i, t: (i, t, 0)),
            pl.BlockSpec((o, c), lambda i, t: (0, 0)),
            pl.BlockSpec((o, 3), lambda i, t: (0, 0)),
            pl.BlockSpec((1, o), lambda i, t: (0, 0)),
        ],
        out_specs=pl.BlockSpec((1, _ZT, o), lambda i, t: (i, t, 0)),
        out_shape=jax.ShapeDtypeStruct((b, n, o), jnp.float32),
    )(features, points_xyz, w0f, w0x, b0.reshape(1, o))
    return z.reshape(b * n, o)


_GW = 256


def _sc_gather(table, flat_idx):
    num = flat_idx.shape[0]
    o = table.shape[1]
    idx2 = flat_idx.reshape(1, num)
    mesh = plsc.VectorSubcoreMesh(core_axis_name="c", subcore_axis_name="s")

    @functools.partial(
        pl.kernel,
        out_type=jax.ShapeDtypeStruct((num, o), table.dtype),
        mesh=mesh,
    )
    def gather_kernel(x_hbm, i_hbm, o_hbm):
        def body(i_vmem, o_vmem):
            pltpu.sync_copy(x_hbm.at[i_vmem.at[0]], o_vmem)

        pltpu.emit_pipeline(
            body,
            grid=(num // _GW,),
            in_specs=[pl.BlockSpec((1, _GW), index_map=lambda i: (0, i))],
            out_specs=[pl.BlockSpec((_GW, o), index_map=lambda i: (i, 0))],
            core_axis_name=("c", "s"),
            dimension_semantics=(pltpu.PARALLEL,),
        )(i_hbm, o_hbm)

    return gather_kernel(table, idx2)


_MT = 64


def _mlp_kernel(g_ref, nxyz_ref, w0x_ref, w1t_ref, b1_ref, w2t_ref, b2_ref, out_ref):
    o0 = g_ref.shape[1]
    o2 = w2t_ref.shape[1]
    cterm = jax.lax.dot_general(
        nxyz_ref[0], w0x_ref[...], (((1,), (1,)), ((), ())),
        preferred_element_type=jnp.float32,
    )
    g3 = g_ref[...].reshape(_MT, K, o0)
    h0 = jnp.maximum(
        g3 - jnp.broadcast_to(cterm.reshape(_MT, 1, o0), (_MT, K, o0)), 0.0
    )
    h0f = h0.reshape(_MT * K, o0).astype(jnp.bfloat16)
    h1 = jnp.maximum(
        jax.lax.dot_general(
            h0f, w1t_ref[...], (((1,), (0,)), ((), ())),
            preferred_element_type=jnp.float32,
        )
        + b1_ref[...],
        0.0,
    ).astype(jnp.bfloat16)
    h2 = jnp.maximum(
        jax.lax.dot_general(
            h1, w2t_ref[...], (((1,), (0,)), ((), ())),
            preferred_element_type=jnp.float32,
        )
        + b2_ref[...],
        0.0,
    )
    pooled = jnp.max(h2.reshape(_MT, K, o2), axis=1)
    out_ref[0] = pooled


def _mlp_call(gathered, new_xyz, W0, W1, b1, W2, b2):
    b = new_xyz.shape[0]
    o0 = W0.shape[0]
    o1 = W1.shape[0]
    o2 = W2.shape[0]
    w0x = W0[:, :3]
    w1t = W1.T.astype(jnp.bfloat16)
    w2t = W2.T.astype(jnp.bfloat16)
    tiles = NPOINT // _MT
    pooled = pl.pallas_call(
        _mlp_kernel,
        grid=(b, tiles),
        in_specs=[
            pl.BlockSpec((_MT * K, o0), lambda i, t: (i * tiles + t, 0)),
            pl.BlockSpec((1, _MT, 3), lambda i, t: (i, t, 0)),
            pl.BlockSpec((o0, 3), lambda i, t: (0, 0)),
            pl.BlockSpec((o0, o1), lambda i, t: (0, 0)),
            pl.BlockSpec((1, o1), lambda i, t: (0, 0)),
            pl.BlockSpec((o1, o2), lambda i, t: (0, 0)),
            pl.BlockSpec((1, o2), lambda i, t: (0, 0)),
        ],
        out_specs=pl.BlockSpec((1, _MT, o2), lambda i, t: (i, t, 0)),
        out_shape=jax.ShapeDtypeStruct((b, NPOINT, o2), jnp.float32),
    )(
        gathered,
        new_xyz,
        w0x,
        w1t,
        b1.reshape(1, o1),
        w2t,
        b2.reshape(1, o2),
    )
    return pooled


def kernel(points_xyz, features, W0, b0, W1, b1, W2, b2):
    b, n, _ = points_xyz.shape
    xyz_t = jnp.transpose(points_xyz, (0, 2, 1))
    idx, new_xyz = _fps_call(points_xyz, xyz_t)
    gidx = _ballquery_call(points_xyz, new_xyz, xyz_t)
    z = _buildz_call(features, points_xyz, W0, b0)
    offs = (jnp.arange(b, dtype=jnp.int32) * n).reshape(b, 1, 1)
    flat_idx = (gidx + offs).reshape(b * NPOINT * K)
    gathered = _sc_gather(z, flat_idx)
    pooled = _mlp_call(gathered, new_xyz, W0, W1, b1, W2, b2)
    new_features = jnp.transpose(pooled, (0, 2, 1))
    return (new_xyz, new_features, idx)
